# Initial kernel scaffold; baseline (speedup 1.0000x reference)
import jax
import jax.numpy as jnp
from jax.experimental import pallas as pl


def kernel(x, edges, pos_edges, neg_edges, eps, W_enc, b_enc, W_vae, b_vae):
    raise NotImplementedError("write your pallas kernel here")


# speedup vs baseline: 1.6995x; 1.6995x over previous
import jax
import jax.numpy as jnp
from jax import lax
from jax.experimental import pallas as pl
from jax.experimental.pallas import tpu as pltpu
from jax.experimental.pallas import tpu_sc as plsc

_EPS = 1e-15
_MAX_LOGSTD = 10.0
_N = 10000
_E = 320000
_D = 128
_H = 128

_NC = 2
_NS = 16
_NW = _NC * _NS
_L = 16

_EW = _E // _NW
_C = 80
_NCH = _EW // _C

_NHALF = 5120
_NTRASH = 128
_NSH = _NHALF + _NTRASH
_RPT = _NSH // _NS
_WPT = _NHALF // _NS
_NP = 2 * _NHALF
_ET = _E // _NS
_NPH = 2
_NCHP = _ET // (_NPH * _C)


def _agg_kernel(x_hbm, row_hbm, col_hbm, out_hbm,
                rowi, coli, buf0, buf1, shared, sem0, sem1):
    cid = lax.axis_index("c")
    sid = lax.axis_index("s")

    zeros = jnp.zeros((_L,), jnp.float32)

    def zrow(r, _):
        for c in range(_H // _L):
            buf0[r, pl.ds(c * _L, _L)] = zeros
        return 0

    lax.fori_loop(0, _C, zrow, 0)
    zbase = sid * _RPT
    for q in range(_RPT // _C):
        pltpu.sync_copy(buf0, shared.at[pl.ds(zbase + q * _C, _C)])
    _ZTAIL = _RPT % _C
    pltpu.sync_copy(buf0.at[pl.ds(0, _ZTAIL)],
                    shared.at[pl.ds(zbase + (_RPT // _C) * _C, _ZTAIL)])
    plsc.subcore_barrier()

    base = cid * _NHALF
    lane = jnp.arange(_L, dtype=jnp.int32)

    def adjust_rows(_p):
        def arow(r, _):
            for v in range(_C // _L):
                r16 = rowi[r, pl.ds(v * _L, _L)]
                adj = r16 - base
                ok = (adj >= 0) & (adj < _NHALF)
                trash = _NHALF + ((lane + r * _C + v * _L) & (_NTRASH - 1))
                rowi[r, pl.ds(v * _L, _L)] = jnp.where(ok, adj, trash)
            return 0

        lax.fori_loop(0, _NCHP, arow, 0)

    def gather(j, buf, sem):
        return pltpu.async_copy(x_hbm.at[coli.at[j]], buf, sem)

    def gwait(j, buf, sem):
        pltpu.make_async_copy(x_hbm.at[coli.at[j]], buf, sem).wait()

    def scatter(j, buf):
        pltpu.sync_copy(buf, shared.at[rowi.at[j]], add=True)

    for p in range(_NPH):
        pltpu.sync_copy(row_hbm.at[sid, p], rowi)
        pltpu.sync_copy(col_hbm.at[sid, p], coli)
        adjust_rows(p)

        gather(0, buf0, sem0)

        def step(k, _):
            j0 = 2 * k
            gather(j0 + 1, buf1, sem1)
            gwait(j0, buf0, sem0)
            scatter(j0, buf0)
            gather(j0 + 2, buf0, sem0)
            gwait(j0 + 1, buf1, sem1)
            scatter(j0 + 1, buf1)
            return 0

        lax.fori_loop(0, (_NCHP - 1) // 2, step, 0)
        gwait(_NCHP - 1, buf0, sem0)
        scatter(_NCHP - 1, buf0)

    plsc.subcore_barrier()
    pltpu.sync_copy(shared.at[pl.ds(sid * _WPT, _WPT)],
                    out_hbm.at[pl.ds(cid * _NHALF + sid * _WPT, _WPT)])


def _dots_chunk(j, abuf, bbuf, dots):
    base = j * _C
    eye = jnp.arange(_L, dtype=jnp.int32)
    for g in range(_C // _L):
        rows = eye + g * _L

        def col_step(t, acc):
            for u in range(8):
                cidx = jnp.full((_L,), t * 8 + u, jnp.int32)
                av = plsc.load_gather(abuf, [rows, cidx])
                bv = plsc.load_gather(bbuf, [rows, cidx])
                acc = acc + av * bv
            return acc

        acc = lax.fori_loop(0, _H // 8, col_step, jnp.zeros((_L,), jnp.float32))
        dots[pl.ds(base + g * _L, _L)] = acc


def _dec_kernel(z_hbm, ps_hbm, pd_hbm, ns_hbm, nd_hbm, pout_hbm, nout_hbm,
                si, di, a0, b0, a1, b1, dots, sa0, sb0, sa1, sb1):
    cid = lax.axis_index("c")
    sid = lax.axis_index("s")
    wid = sid * _NC + cid

    def run_list(s_hbm, d_hbm, out_hbm):
        pltpu.sync_copy(s_hbm.at[wid], si)
        pltpu.sync_copy(d_hbm.at[wid], di)

        def issue(j, abuf, bbuf, sa, sb):
            pltpu.async_copy(z_hbm.at[si.at[j]], abuf, sa)
            pltpu.async_copy(z_hbm.at[di.at[j]], bbuf, sb)

        def drain(j, abuf, bbuf, sa, sb):
            pltpu.make_async_copy(z_hbm.at[si.at[j]], abuf, sa).wait()
            pltpu.make_async_copy(z_hbm.at[di.at[j]], bbuf, sb).wait()

        issue(0, a0, b0, sa0, sb0)

        def step(k, _):
            j0 = 2 * k
            issue(j0 + 1, a1, b1, sa1, sb1)
            drain(j0, a0, b0, sa0, sb0)
            _dots_chunk(j0, a0, b0, dots)
            issue(j0 + 2, a0, b0, sa0, sb0)
            drain(j0 + 1, a1, b1, sa1, sb1)
            _dots_chunk(j0 + 1, a1, b1, dots)
            return 0

        lax.fori_loop(0, (_NCH - 1) // 2, step, 0)
        drain(_NCH - 1, a0, b0, sa0, sb0)
        _dots_chunk(_NCH - 1, a0, b0, dots)
        pltpu.sync_copy(dots, out_hbm.at[wid])

    run_list(ps_hbm, pd_hbm, pout_hbm)
    run_list(ns_hbm, nd_hbm, nout_hbm)


_R = 1000


def _enc_kernel(p_ref, eps_ref, we_ref, be_ref, wv_ref, bv_ref,
                z_ref, kl_ref):
    i = pl.program_id(0)
    agg = p_ref[...]
    h = jnp.maximum(
        jnp.dot(agg, we_ref[...], preferred_element_type=jnp.float32)
        + be_ref[...], 0.0)
    out = (jnp.dot(h, wv_ref[...], preferred_element_type=jnp.float32)
           + bv_ref[...])
    mu = out[:, :_H]
    ls = jnp.minimum(out[:, _H:], _MAX_LOGSTD)
    els = jnp.exp(ls)
    z_ref[...] = mu + eps_ref[...] * els
    klb = jnp.sum(1.0 + 2.0 * ls - mu * mu - els * els)

    @pl.when(i == 0)
    def _():
        kl_ref[0, 0] = 0.0

    kl_ref[0, 0] += klb


def _loss_kernel(pd_ref, nd_ref, kl_ref, out_ref):
    pv = 1.0 / (1.0 + jnp.exp(-pd_ref[...]))
    nv = 1.0 / (1.0 + jnp.exp(-nd_ref[...]))
    pos_loss = jnp.sum(-jnp.log(pv + _EPS)) / _E
    neg_loss = jnp.sum(-jnp.log(1.0 - nv + _EPS)) / _E
    kl = (-0.5 / (_N * _N)) * kl_ref[0, 0]
    out_ref[0, 0] = pos_loss + neg_loss + kl


def _sc_aggregate(x, row4, col4):
    return pl.kernel(
        _agg_kernel,
        out_type=jax.ShapeDtypeStruct((_NP, _H), jnp.float32),
        mesh=plsc.VectorSubcoreMesh(core_axis_name="c", subcore_axis_name="s"),
        compiler_params=pltpu.CompilerParams(needs_layout_passes=False),
        scratch_types=[
            pltpu.VMEM((_NCHP, _C), jnp.int32),
            pltpu.VMEM((_NCHP, _C), jnp.int32),
            pltpu.VMEM((_C, _H), jnp.float32),
            pltpu.VMEM((_C, _H), jnp.float32),
            pltpu.VMEM_SHARED((_NSH, _H), jnp.float32),
            pltpu.SemaphoreType.DMA,
            pltpu.SemaphoreType.DMA,
        ],
    )(x, row4, col4)


def _sc_decode(z, ps3, pd3, ns3, nd3):
    return pl.kernel(
        _dec_kernel,
        out_type=(jax.ShapeDtypeStruct((_NW, _EW), jnp.float32),
                  jax.ShapeDtypeStruct((_NW, _EW), jnp.float32)),
        mesh=plsc.VectorSubcoreMesh(core_axis_name="c", subcore_axis_name="s"),
        compiler_params=pltpu.CompilerParams(needs_layout_passes=False),
        scratch_types=[
            pltpu.VMEM((_NCH, _C), jnp.int32),
            pltpu.VMEM((_NCH, _C), jnp.int32),
            pltpu.VMEM((_C, _H), jnp.float32),
            pltpu.VMEM((_C, _H), jnp.float32),
            pltpu.VMEM((_C, _H), jnp.float32),
            pltpu.VMEM((_C, _H), jnp.float32),
            pltpu.VMEM((_EW,), jnp.float32),
            pltpu.SemaphoreType.DMA,
            pltpu.SemaphoreType.DMA,
            pltpu.SemaphoreType.DMA,
            pltpu.SemaphoreType.DMA,
        ],
    )(z, ps3, pd3, ns3, nd3)


def _tc_encode(parts, eps, W_enc, b_enc, W_vae, b_vae):
    grid = _N // _R
    return pl.pallas_call(
        _enc_kernel,
        grid=(grid,),
        in_specs=[
            pl.BlockSpec((_R, _H), lambda i: (i, 0)),
            pl.BlockSpec((_R, _H), lambda i: (i, 0)),
            pl.BlockSpec((_D, _H), lambda i: (0, 0)),
            pl.BlockSpec((1, _H), lambda i: (0, 0)),
            pl.BlockSpec((_H, 2 * _H), lambda i: (0, 0)),
            pl.BlockSpec((1, 2 * _H), lambda i: (0, 0)),
        ],
        out_specs=[
            pl.BlockSpec((_R, _H), lambda i: (i, 0)),
            pl.BlockSpec(memory_space=pltpu.SMEM),
        ],
        out_shape=[
            jax.ShapeDtypeStruct((_N, _H), jnp.float32),
            jax.ShapeDtypeStruct((1, 1), jnp.float32),
        ],
    )(parts, eps, W_enc, b_enc.reshape(1, _H), W_vae, b_vae.reshape(1, 2 * _H))


def _tc_loss(pos_dots, neg_dots, kl_raw):
    return pl.pallas_call(
        _loss_kernel,
        in_specs=[
            pl.BlockSpec((_E // _D, _D), lambda: (0, 0)),
            pl.BlockSpec((_E // _D, _D), lambda: (0, 0)),
            pl.BlockSpec(memory_space=pltpu.SMEM),
        ],
        out_specs=pl.BlockSpec(memory_space=pltpu.SMEM),
        out_shape=jax.ShapeDtypeStruct((1, 1), jnp.float32),
    )(pos_dots.reshape(_E // _D, _D), neg_dots.reshape(_E // _D, _D), kl_raw)


def kernel(x, edges, pos_edges, neg_edges, eps, W_enc, b_enc, W_vae, b_vae):
    row4 = edges[:, 0].reshape(_NS, _NPH, _NCHP, _C)
    col4 = edges[:, 1].reshape(_NS, _NPH, _NCHP, _C)
    ps3 = pos_edges[:, 0].reshape(_NW, _NCH, _C)
    pd3 = pos_edges[:, 1].reshape(_NW, _NCH, _C)
    ns3 = neg_edges[:, 0].reshape(_NW, _NCH, _C)
    nd3 = neg_edges[:, 1].reshape(_NW, _NCH, _C)

    parts = _sc_aggregate(x, row4, col4)
    z, kl_raw = _tc_encode(parts, eps, W_enc, b_enc, W_vae, b_vae)
    pos_dots, neg_dots = _sc_decode(z, ps3, pd3, ns3, nd3)
    loss = _tc_loss(pos_dots, neg_dots, kl_raw)
    return loss[0, 0]
